# Initial kernel scaffold; baseline (speedup 1.0000x reference)
import jax
import jax.numpy as jnp
from jax.experimental import pallas as pl


def kernel(edge_feats, segment_ids, num_graphs, W, b):
    raise NotImplementedError("write your pallas kernel here")


# speedup vs baseline: 1.6294x; 1.6294x over previous
import functools

import jax
import jax.numpy as jnp
from jax import lax
from jax.experimental import pallas as pl
from jax.experimental.pallas import tpu as pltpu
from jax.experimental.pallas import tpu_sc as plsc

E = 320000
D = 128
G = 256
NC = 2
NS = 16
L = 16
NW = NC * NS
EW = E // NW
C = 80
NCHUNK = EW // C
GROUPS = C // L


def _tanh(z):
    a = jnp.abs(z)
    e = jnp.exp(a * (-2.0))
    return jnp.sign(z) * (1.0 - e) / (1.0 + e)


def _sc_body(feats_hbm, ids_hbm, w_hbm, b_hbm, out_hbm,
             buf, ids_v, acc, wbuf, bbuf, smat):
    wid = lax.axis_index("s") * NC + lax.axis_index("c")
    row0 = wid * EW

    pltpu.sync_copy(w_hbm, wbuf)
    pltpu.sync_copy(b_hbm, bbuf)
    bv = bbuf[...]
    wv = [wbuf[pl.ds(j * L, L)] for j in range(D // L)]

    def _zero(i, _):
        acc[pl.ds(i * L, L)] = jnp.zeros((L,), jnp.float32)
        return 0
    lax.fori_loop(0, G * D // L, _zero, 0)

    def _chunk(ci, _):
        base = row0 + ci * C
        pltpu.sync_copy(feats_hbm.at[pl.ds(base, C)], buf)
        pltpu.sync_copy(ids_hbm.at[pl.ds(base, C)], ids_v)

        lanes = lax.iota(jnp.int32, L)

        def _group(gi, _):
            r0 = gi * L
            for r in range(L):
                row = r0 + r
                s = buf[row, pl.ds(0, L)] * wv[0]
                for j in range(1, D // L):
                    s = s + buf[row, pl.ds(j * L, L)] * wv[j]
                smat[pl.ds(r * L, L)] = s
            hv = plsc.load_gather(smat, [lanes * L])
            for j in range(1, L):
                hv = hv + plsc.load_gather(smat, [lanes * L + j])
            wg = _tanh(hv + bv)
            idv = jnp.minimum(ids_v[pl.ds(r0, L)], G - 1)
            for r in range(L):
                row = r0 + r
                off = idv[r] * D
                wr = wg[r]
                for j in range(D // L):
                    x = buf[row, pl.ds(j * L, L)] * wr
                    plsc.addupdate(acc.at[pl.ds(off + j * L, L)], x)
            return 0

        lax.fori_loop(0, GROUPS, _group, 0)
        return 0

    lax.fori_loop(0, NCHUNK, _chunk, 0)
    pltpu.sync_copy(acc, out_hbm.at[wid])


def _combine_body(parts_ref, o_ref):
    o_ref[...] = jnp.sum(parts_ref[...], axis=0)


@jax.jit
def _run(edge_feats, ids32, w_flat, b_pad):
    mesh = plsc.VectorSubcoreMesh(core_axis_name="c", subcore_axis_name="s",
                                  num_cores=NC, num_subcores=NS)
    sc = pl.kernel(
        _sc_body,
        out_type=jax.ShapeDtypeStruct((NW, G * D), jnp.float32),
        mesh=mesh,
        compiler_params=pltpu.CompilerParams(needs_layout_passes=False),
        scratch_types=[
            pltpu.VMEM((C, D), jnp.float32),
            pltpu.VMEM((C,), jnp.int32),
            pltpu.VMEM((G * D,), jnp.float32),
            pltpu.VMEM((D,), jnp.float32),
            pltpu.VMEM((L,), jnp.float32),
            pltpu.VMEM((L * L,), jnp.float32),
        ],
    )
    parts = sc(edge_feats, ids32, w_flat, b_pad)
    out = pl.pallas_call(
        _combine_body,
        out_shape=jax.ShapeDtypeStruct((G, D), jnp.float32),
    )(parts.reshape(NW, G, D))
    return out


def kernel(edge_feats, segment_ids, num_graphs, W, b):
    ids32 = segment_ids.astype(jnp.int32)
    w_flat = W.reshape(D)
    b_pad = jnp.broadcast_to(b.reshape(1), (L,)).astype(jnp.float32)
    return _run(edge_feats, ids32, w_flat, b_pad)
